# Initial kernel scaffold; baseline (speedup 1.0000x reference)
import jax
import jax.numpy as jnp
from jax.experimental import pallas as pl


def kernel(rbf, species, idx_i, idx_j, embedding_vect, W_rbf, W_concat, b_concat):
    raise NotImplementedError("write your pallas kernel here")


# speedup vs baseline: 7.9026x; 7.9026x over previous
import functools

import jax
import jax.numpy as jnp
from jax import lax
from jax.experimental import pallas as pl
from jax.experimental.pallas import tpu as pltpu
from jax.experimental.pallas import tpu_sc as plsc

N_EDGES = 320000
N_NODES = 10000
N_RBF = 6
EMBED = 128
N_SPECIES = 100

SC_CORES = 2
SC_SUBCORES = 16
SC_LANES = 16
NW = SC_CORES * SC_SUBCORES
CHUNK = N_EDGES // NW

BLK = 2000

@functools.cache
def _make_sc_type_gather():
    mesh = plsc.VectorSubcoreMesh(
        core_axis_name="c", subcore_axis_name="s",
        num_cores=SC_CORES, num_subcores=SC_SUBCORES)

    @functools.partial(
        pl.kernel,
        out_type=(jax.ShapeDtypeStruct((N_EDGES,), jnp.int32),
                  jax.ShapeDtypeStruct((N_EDGES,), jnp.int32)),
        mesh=mesh,
        scratch_types=[pltpu.VMEM((N_NODES,), jnp.int32),
                       pltpu.VMEM((CHUNK,), jnp.int32),
                       pltpu.VMEM((CHUNK,), jnp.int32)],
        compiler_params=pltpu.CompilerParams(needs_layout_passes=False),
    )
    def _sc_type_gather(species_hbm, idx_i_hbm, idx_j_hbm, type_i_hbm,
                        type_j_hbm, species_v, idx_v, out_v):
        wid = lax.axis_index("s") * SC_CORES + lax.axis_index("c")
        base = wid * CHUNK
        pltpu.sync_copy(species_hbm, species_v)
        for src_hbm, dst_hbm in ((idx_i_hbm, type_i_hbm),
                                 (idx_j_hbm, type_j_hbm)):
            pltpu.sync_copy(src_hbm.at[pl.ds(base, CHUNK)], idx_v)

            def body(k, _):
                iv = idx_v[pl.ds(k * SC_LANES, SC_LANES)]
                out_v[pl.ds(k * SC_LANES, SC_LANES)] = plsc.load_gather(
                    species_v, [iv])
                return 0

            lax.fori_loop(0, CHUNK // SC_LANES, body, 0)
            pltpu.sync_copy(out_v, dst_hbm.at[pl.ds(base, CHUNK)])

    return _sc_type_gather


def _tc_body(ti_ref, tj_ref, rbf_ref, Ti_ref, Tj_ref, Wf_ref, b_ref,
             out_ref):
    lane = lax.broadcasted_iota(jnp.int32, (BLK, EMBED), 1)
    oh_i = (ti_ref[...] == lane).astype(jnp.float32)
    oh_j = (tj_ref[...] == lane).astype(jnp.float32)
    acc = jnp.dot(oh_i, Ti_ref[...], preferred_element_type=jnp.float32)
    acc = acc + jnp.dot(oh_j, Tj_ref[...], preferred_element_type=jnp.float32)
    acc = acc + jnp.dot(rbf_ref[...], Wf_ref[...],
                        preferred_element_type=jnp.float32)
    acc = acc + b_ref[...]
    out_ref[...] = acc * jax.nn.sigmoid(acc)


def _tc_fused(type_i, type_j, rbf, Ti, Tj, Wf, b):
    grid = (N_EDGES // BLK,)
    return pl.pallas_call(
        _tc_body,
        grid=grid,
        in_specs=[
            pl.BlockSpec((BLK, 1), lambda i: (i, 0)),
            pl.BlockSpec((BLK, 1), lambda i: (i, 0)),
            pl.BlockSpec((BLK, N_RBF), lambda i: (i, 0)),
            pl.BlockSpec((EMBED, EMBED), lambda i: (0, 0)),
            pl.BlockSpec((EMBED, EMBED), lambda i: (0, 0)),
            pl.BlockSpec((N_RBF, EMBED), lambda i: (0, 0)),
            pl.BlockSpec((1, EMBED), lambda i: (0, 0)),
        ],
        out_specs=pl.BlockSpec((BLK, EMBED), lambda i: (i, 0)),
        out_shape=jax.ShapeDtypeStruct((N_EDGES, EMBED), jnp.float32),
        compiler_params=pltpu.CompilerParams(
            dimension_semantics=("parallel",)),
    )(type_i, type_j, rbf, Ti, Tj, Wf, b)


def kernel(rbf, species, idx_i, idx_j, embedding_vect, W_rbf, W_concat,
           b_concat):
    species = species.astype(jnp.int32)
    idx_i = idx_i.astype(jnp.int32)
    idx_j = idx_j.astype(jnp.int32)

    type_i, type_j = _make_sc_type_gather()(species, idx_i, idx_j)

    half = W_concat.shape[0] // 4
    Ti = jnp.pad(embedding_vect @ W_concat[:half],
                 ((0, EMBED - N_SPECIES), (0, 0)))
    Tj = jnp.pad(embedding_vect @ W_concat[half:2 * half],
                 ((0, EMBED - N_SPECIES), (0, 0)))
    Wf = W_rbf @ W_concat[2 * half:]

    return _tc_fused(type_i.reshape(N_EDGES, 1), type_j.reshape(N_EDGES, 1),
                     rbf, Ti, Tj, Wf, b_concat.reshape(1, EMBED))


# speedup vs baseline: 17.4255x; 2.2050x over previous
import functools

import jax
import jax.numpy as jnp
from jax import lax
from jax.experimental import pallas as pl
from jax.experimental.pallas import tpu as pltpu
from jax.experimental.pallas import tpu_sc as plsc

N_EDGES = 320000
N_NODES = 10000
N_RBF = 6
EMBED = 128
N_SPECIES = 100

SC_CORES = 2
SC_SUBCORES = 16
SC_LANES = 16
NW = SC_CORES * SC_SUBCORES
CHUNK = N_EDGES // NW

GROUP = 128
ROWS = 20
BLK = ROWS * GROUP
N_GROUPS = N_EDGES // GROUP
GRID = N_EDGES // BLK

@functools.cache
def _make_sc_type_gather():
    mesh = plsc.VectorSubcoreMesh(
        core_axis_name="c", subcore_axis_name="s",
        num_cores=SC_CORES, num_subcores=SC_SUBCORES)

    @functools.partial(
        pl.kernel,
        out_type=(jax.ShapeDtypeStruct((N_EDGES,), jnp.int32),
                  jax.ShapeDtypeStruct((N_EDGES,), jnp.int32)),
        mesh=mesh,
        scratch_types=[pltpu.VMEM((N_NODES,), jnp.int32),
                       pltpu.VMEM((CHUNK,), jnp.int32),
                       pltpu.VMEM((CHUNK,), jnp.int32)],
        compiler_params=pltpu.CompilerParams(needs_layout_passes=False),
    )
    def _sc_type_gather(species_hbm, idx_i_hbm, idx_j_hbm, type_i_hbm,
                        type_j_hbm, species_v, idx_v, out_v):
        wid = lax.axis_index("s") * SC_CORES + lax.axis_index("c")
        base = wid * CHUNK
        pltpu.sync_copy(species_hbm, species_v)
        for src_hbm, dst_hbm in ((idx_i_hbm, type_i_hbm),
                                 (idx_j_hbm, type_j_hbm)):
            pltpu.sync_copy(src_hbm.at[pl.ds(base, CHUNK)], idx_v)

            def body(k, _):
                iv = idx_v[pl.ds(k * SC_LANES, SC_LANES)]
                out_v[pl.ds(k * SC_LANES, SC_LANES)] = plsc.load_gather(
                    species_v, [iv])
                return 0

            lax.fori_loop(0, CHUNK // SC_LANES, body, 0)
            pltpu.sync_copy(out_v, dst_hbm.at[pl.ds(base, CHUNK)])

    return _sc_type_gather


def _tc_body(ti_ref, tj_ref, rbf_ref, Ti_ref, Tj_ref, Wf_ref, b_ref,
             out_ref):
    cls = lax.broadcasted_iota(jnp.int32, (EMBED, GROUP), 0)
    ohT_i = jnp.concatenate(
        [(ti_ref[0, r:r + 1, :] == cls).astype(jnp.float32)
         for r in range(ROWS)], axis=1)
    ohT_j = jnp.concatenate(
        [(tj_ref[0, r:r + 1, :] == cls).astype(jnp.float32)
         for r in range(ROWS)], axis=1)
    dnums = (((0,), (0,)), ((), ()))
    acc = lax.dot_general(ohT_i, Ti_ref[...], dnums,
                          preferred_element_type=jnp.float32)
    acc = acc + lax.dot_general(ohT_j, Tj_ref[...], dnums,
                                preferred_element_type=jnp.float32)
    acc = acc + jnp.dot(rbf_ref[...], Wf_ref[...],
                        preferred_element_type=jnp.float32)
    acc = acc + b_ref[...]
    out_ref[...] = acc * jax.nn.sigmoid(acc)


def _tc_fused(type_i, type_j, rbf, Ti, Tj, Wf, b):
    return pl.pallas_call(
        _tc_body,
        grid=(GRID,),
        in_specs=[
            pl.BlockSpec((1, ROWS, GROUP), lambda i: (i, 0, 0)),
            pl.BlockSpec((1, ROWS, GROUP), lambda i: (i, 0, 0)),
            pl.BlockSpec((BLK, N_RBF), lambda i: (i, 0)),
            pl.BlockSpec((EMBED, EMBED), lambda i: (0, 0)),
            pl.BlockSpec((EMBED, EMBED), lambda i: (0, 0)),
            pl.BlockSpec((N_RBF, EMBED), lambda i: (0, 0)),
            pl.BlockSpec((1, EMBED), lambda i: (0, 0)),
        ],
        out_specs=pl.BlockSpec((BLK, EMBED), lambda i: (i, 0)),
        out_shape=jax.ShapeDtypeStruct((N_EDGES, EMBED), jnp.float32),
        compiler_params=pltpu.CompilerParams(
            dimension_semantics=("parallel",)),
    )(type_i, type_j, rbf, Ti, Tj, Wf, b)


def kernel(rbf, species, idx_i, idx_j, embedding_vect, W_rbf, W_concat,
           b_concat):
    species = species.astype(jnp.int32)
    idx_i = idx_i.astype(jnp.int32)
    idx_j = idx_j.astype(jnp.int32)

    type_i, type_j = _make_sc_type_gather()(species, idx_i, idx_j)

    half = W_concat.shape[0] // 4
    Ti = jnp.pad(embedding_vect @ W_concat[:half],
                 ((0, EMBED - N_SPECIES), (0, 0)))
    Tj = jnp.pad(embedding_vect @ W_concat[half:2 * half],
                 ((0, EMBED - N_SPECIES), (0, 0)))
    Wf = W_rbf @ W_concat[2 * half:]

    return _tc_fused(type_i.reshape(GRID, ROWS, GROUP),
                     type_j.reshape(GRID, ROWS, GROUP),
                     rbf, Ti, Tj, Wf, b_concat.reshape(1, EMBED))


# speedup vs baseline: 27.8598x; 1.5988x over previous
import functools

import jax
import jax.numpy as jnp
from jax import lax
from jax.experimental import pallas as pl
from jax.experimental.pallas import tpu as pltpu
from jax.experimental.pallas import tpu_sc as plsc

N_EDGES = 320000
N_NODES = 10000
N_RBF = 6
EMBED = 128
N_SPECIES = 100

SC_CORES = 2
SC_SUBCORES = 16
SC_LANES = 16
NW = SC_CORES * SC_SUBCORES
CHUNK = N_EDGES // NW

GROUP = 128
ROWS = 20
BLK = ROWS * GROUP
N_GROUPS = N_EDGES // GROUP
GRID = N_EDGES // BLK

@functools.cache
def _make_sc_type_gather():
    mesh = plsc.VectorSubcoreMesh(
        core_axis_name="c", subcore_axis_name="s",
        num_cores=SC_CORES, num_subcores=SC_SUBCORES)

    @functools.partial(
        pl.kernel,
        out_type=(jax.ShapeDtypeStruct((N_EDGES,), jnp.int32),
                  jax.ShapeDtypeStruct((N_EDGES,), jnp.int32)),
        mesh=mesh,
        scratch_types=[pltpu.VMEM((N_NODES,), jnp.int32),
                       pltpu.VMEM((CHUNK,), jnp.int32),
                       pltpu.VMEM((CHUNK,), jnp.int32)],
        compiler_params=pltpu.CompilerParams(needs_layout_passes=False),
    )
    def _sc_type_gather(species_hbm, idx_i_hbm, idx_j_hbm, type_i_hbm,
                        type_j_hbm, species_v, idx_v, out_v):
        wid = lax.axis_index("s") * SC_CORES + lax.axis_index("c")
        base = wid * CHUNK
        pltpu.sync_copy(species_hbm, species_v)
        for src_hbm, dst_hbm in ((idx_i_hbm, type_i_hbm),
                                 (idx_j_hbm, type_j_hbm)):
            pltpu.sync_copy(src_hbm.at[pl.ds(base, CHUNK)], idx_v)

            def body(k, _):
                iv = idx_v[pl.ds(k * SC_LANES, SC_LANES)]
                out_v[pl.ds(k * SC_LANES, SC_LANES)] = plsc.load_gather(
                    species_v, [iv])
                return 0

            lax.fori_loop(0, CHUNK // SC_LANES, body, 0)
            pltpu.sync_copy(out_v, dst_hbm.at[pl.ds(base, CHUNK)])

    return _sc_type_gather


def _tc_body(ti_ref, tj_ref, rbf_ref, Ti_ref, Tj_ref, Wf_ref, b_ref,
             out_ref):
    cls = lax.broadcasted_iota(jnp.int32, (EMBED, GROUP), 0)
    ohT_i = jnp.concatenate(
        [(ti_ref[0, r:r + 1, :] == cls).astype(jnp.float32)
         for r in range(ROWS)], axis=1)
    ohT_j = jnp.concatenate(
        [(tj_ref[0, r:r + 1, :] == cls).astype(jnp.float32)
         for r in range(ROWS)], axis=1)
    dnums = (((0,), (0,)), ((), ()))
    acc = lax.dot_general(ohT_i, Ti_ref[...], dnums,
                          preferred_element_type=jnp.float32)
    acc = acc + lax.dot_general(ohT_j, Tj_ref[...], dnums,
                                preferred_element_type=jnp.float32)
    acc = acc + lax.dot_general(rbf_ref[...], Wf_ref[...], dnums,
                                preferred_element_type=jnp.float32)
    acc = acc + b_ref[...]
    out_ref[...] = acc * jax.nn.sigmoid(acc)


def _tc_fused(type_i, type_j, rbf, Ti, Tj, Wf, b):
    return pl.pallas_call(
        _tc_body,
        grid=(GRID,),
        in_specs=[
            pl.BlockSpec((1, ROWS, GROUP), lambda i: (i, 0, 0)),
            pl.BlockSpec((1, ROWS, GROUP), lambda i: (i, 0, 0)),
            pl.BlockSpec((N_RBF, BLK), lambda i: (0, i)),
            pl.BlockSpec((EMBED, EMBED), lambda i: (0, 0)),
            pl.BlockSpec((EMBED, EMBED), lambda i: (0, 0)),
            pl.BlockSpec((N_RBF, EMBED), lambda i: (0, 0)),
            pl.BlockSpec((1, EMBED), lambda i: (0, 0)),
        ],
        out_specs=pl.BlockSpec((BLK, EMBED), lambda i: (i, 0)),
        out_shape=jax.ShapeDtypeStruct((N_EDGES, EMBED), jnp.float32),
        compiler_params=pltpu.CompilerParams(
            dimension_semantics=("parallel",)),
    )(type_i, type_j, rbf, Ti, Tj, Wf, b)


def kernel(rbf, species, idx_i, idx_j, embedding_vect, W_rbf, W_concat,
           b_concat):
    species = species.astype(jnp.int32)
    idx_i = idx_i.astype(jnp.int32)
    idx_j = idx_j.astype(jnp.int32)

    type_i, type_j = _make_sc_type_gather()(species, idx_i, idx_j)

    half = W_concat.shape[0] // 4
    Ti = jnp.pad(embedding_vect @ W_concat[:half],
                 ((0, EMBED - N_SPECIES), (0, 0)))
    Tj = jnp.pad(embedding_vect @ W_concat[half:2 * half],
                 ((0, EMBED - N_SPECIES), (0, 0)))
    Wf = W_rbf @ W_concat[2 * half:]

    return _tc_fused(type_i.reshape(GRID, ROWS, GROUP),
                     type_j.reshape(GRID, ROWS, GROUP),
                     rbf.T, Ti, Tj, Wf, b_concat.reshape(1, EMBED))


# speedup vs baseline: 28.4326x; 1.0206x over previous
import functools

import jax
import jax.numpy as jnp
from jax import lax
from jax.experimental import pallas as pl
from jax.experimental.pallas import tpu as pltpu
from jax.experimental.pallas import tpu_sc as plsc

N_EDGES = 320000
N_NODES = 10000
N_RBF = 6
EMBED = 128
N_SPECIES = 100

SC_CORES = 2
SC_SUBCORES = 16
SC_LANES = 16
NW = SC_CORES * SC_SUBCORES
CHUNK = N_EDGES // NW

GROUP = 128
ROWS = 20
BLK = ROWS * GROUP
N_GROUPS = N_EDGES // GROUP
GRID = N_EDGES // BLK

@functools.cache
def _make_sc_type_gather():
    mesh = plsc.VectorSubcoreMesh(
        core_axis_name="c", subcore_axis_name="s",
        num_cores=SC_CORES, num_subcores=SC_SUBCORES)

    @functools.partial(
        pl.kernel,
        out_type=(jax.ShapeDtypeStruct((N_EDGES,), jnp.int32),
                  jax.ShapeDtypeStruct((N_EDGES,), jnp.int32)),
        mesh=mesh,
        scratch_types=[pltpu.VMEM((N_NODES,), jnp.int32),
                       pltpu.VMEM((CHUNK,), jnp.int32),
                       pltpu.VMEM((CHUNK,), jnp.int32)],
        compiler_params=pltpu.CompilerParams(needs_layout_passes=False),
    )
    def _sc_type_gather(species_hbm, idx_i_hbm, idx_j_hbm, type_i_hbm,
                        type_j_hbm, species_v, idx_v, out_v):
        wid = lax.axis_index("s") * SC_CORES + lax.axis_index("c")
        base = wid * CHUNK
        pltpu.sync_copy(species_hbm, species_v)
        for src_hbm, dst_hbm in ((idx_i_hbm, type_i_hbm),
                                 (idx_j_hbm, type_j_hbm)):
            pltpu.sync_copy(src_hbm.at[pl.ds(base, CHUNK)], idx_v)

            def body(k, _):
                iv = idx_v[pl.ds(k * SC_LANES, SC_LANES)]
                out_v[pl.ds(k * SC_LANES, SC_LANES)] = plsc.load_gather(
                    species_v, [iv])
                return 0

            lax.fori_loop(0, CHUNK // SC_LANES, body, 0)
            pltpu.sync_copy(out_v, dst_hbm.at[pl.ds(base, CHUNK)])

    return _sc_type_gather


def _tc_body(ti_ref, tj_ref, rbf_ref, Ti_ref, Tj_ref, Wf_ref, b_ref,
             out_ref):
    cls = lax.broadcasted_iota(jnp.int32, (EMBED, GROUP), 0)
    ohT_i = jnp.concatenate(
        [(ti_ref[0, r:r + 1, :] == cls).astype(jnp.bfloat16)
         for r in range(ROWS)], axis=1)
    ohT_j = jnp.concatenate(
        [(tj_ref[0, r:r + 1, :] == cls).astype(jnp.bfloat16)
         for r in range(ROWS)], axis=1)
    dnums = (((0,), (0,)), ((), ()))
    acc = lax.dot_general(ohT_i, Ti_ref[...], dnums,
                          preferred_element_type=jnp.float32)
    acc = acc + lax.dot_general(ohT_j, Tj_ref[...], dnums,
                                preferred_element_type=jnp.float32)
    acc = acc + lax.dot_general(rbf_ref[...].astype(jnp.bfloat16),
                                Wf_ref[...], dnums,
                                preferred_element_type=jnp.float32)
    acc = acc + b_ref[...]
    out_ref[...] = acc * jax.nn.sigmoid(acc)


def _tc_fused(type_i, type_j, rbf, Ti, Tj, Wf, b):
    return pl.pallas_call(
        _tc_body,
        grid=(GRID,),
        in_specs=[
            pl.BlockSpec((1, ROWS, GROUP), lambda i: (i, 0, 0)),
            pl.BlockSpec((1, ROWS, GROUP), lambda i: (i, 0, 0)),
            pl.BlockSpec((N_RBF, BLK), lambda i: (0, i)),
            pl.BlockSpec((EMBED, EMBED), lambda i: (0, 0)),
            pl.BlockSpec((EMBED, EMBED), lambda i: (0, 0)),
            pl.BlockSpec((N_RBF, EMBED), lambda i: (0, 0)),
            pl.BlockSpec((1, EMBED), lambda i: (0, 0)),
        ],
        out_specs=pl.BlockSpec((BLK, EMBED), lambda i: (i, 0)),
        out_shape=jax.ShapeDtypeStruct((N_EDGES, EMBED), jnp.float32),
        compiler_params=pltpu.CompilerParams(
            dimension_semantics=("parallel",)),
    )(type_i, type_j, rbf, Ti, Tj, Wf, b)


def kernel(rbf, species, idx_i, idx_j, embedding_vect, W_rbf, W_concat,
           b_concat):
    species = species.astype(jnp.int32)
    idx_i = idx_i.astype(jnp.int32)
    idx_j = idx_j.astype(jnp.int32)

    type_i, type_j = _make_sc_type_gather()(species, idx_i, idx_j)

    half = W_concat.shape[0] // 4
    Ti = jnp.pad(embedding_vect @ W_concat[:half],
                 ((0, EMBED - N_SPECIES), (0, 0))).astype(jnp.bfloat16)
    Tj = jnp.pad(embedding_vect @ W_concat[half:2 * half],
                 ((0, EMBED - N_SPECIES), (0, 0))).astype(jnp.bfloat16)
    Wf = (W_rbf @ W_concat[2 * half:]).astype(jnp.bfloat16)

    return _tc_fused(type_i.reshape(GRID, ROWS, GROUP),
                     type_j.reshape(GRID, ROWS, GROUP),
                     rbf.T, Ti, Tj, Wf, b_concat.reshape(1, EMBED))


# speedup vs baseline: 36.9475x; 1.2995x over previous
import functools

import jax
import jax.numpy as jnp
from jax import lax
from jax.experimental import pallas as pl
from jax.experimental.pallas import tpu as pltpu
from jax.experimental.pallas import tpu_sc as plsc

N_EDGES = 320000
N_NODES = 10000
N_RBF = 6
EMBED = 128
N_SPECIES = 100

SC_CORES = 2
SC_SUBCORES = 16
SC_LANES = 16
NW = SC_CORES * SC_SUBCORES
CHUNK = N_EDGES // NW

GROUP = 128
ROWS = 50
BLK = ROWS * GROUP
N_GROUPS = N_EDGES // GROUP
GRID = N_EDGES // BLK

@functools.cache
def _make_sc_type_gather():
    mesh = plsc.VectorSubcoreMesh(
        core_axis_name="c", subcore_axis_name="s",
        num_cores=SC_CORES, num_subcores=SC_SUBCORES)

    @functools.partial(
        pl.kernel,
        out_type=(jax.ShapeDtypeStruct((N_EDGES,), jnp.int32),
                  jax.ShapeDtypeStruct((N_EDGES,), jnp.int32)),
        mesh=mesh,
        scratch_types=[pltpu.VMEM((N_NODES,), jnp.int32),
                       pltpu.VMEM((CHUNK,), jnp.int32),
                       pltpu.VMEM((CHUNK,), jnp.int32)],
        compiler_params=pltpu.CompilerParams(needs_layout_passes=False),
    )
    def _sc_type_gather(species_hbm, idx_i_hbm, idx_j_hbm, type_i_hbm,
                        type_j_hbm, species_v, idx_v, out_v):
        wid = lax.axis_index("s") * SC_CORES + lax.axis_index("c")
        base = wid * CHUNK
        pltpu.sync_copy(species_hbm, species_v)
        for src_hbm, dst_hbm in ((idx_i_hbm, type_i_hbm),
                                 (idx_j_hbm, type_j_hbm)):
            pltpu.sync_copy(src_hbm.at[pl.ds(base, CHUNK)], idx_v)

            def body(k, _):
                iv = idx_v[pl.ds(k * SC_LANES, SC_LANES)]
                out_v[pl.ds(k * SC_LANES, SC_LANES)] = plsc.load_gather(
                    species_v, [iv])
                return 0

            lax.fori_loop(0, CHUNK // SC_LANES, body, 0)
            pltpu.sync_copy(out_v, dst_hbm.at[pl.ds(base, CHUNK)])

    return _sc_type_gather


def _tc_body(ti_ref, tj_ref, rbf_ref, Ti_ref, Tj_ref, Wf_ref, b_ref,
             out_ref):
    cls = lax.broadcasted_iota(jnp.int32, (EMBED, GROUP), 0)
    ohT_i = jnp.concatenate(
        [(ti_ref[0, r:r + 1, :] == cls).astype(jnp.bfloat16)
         for r in range(ROWS)], axis=1)
    ohT_j = jnp.concatenate(
        [(tj_ref[0, r:r + 1, :] == cls).astype(jnp.bfloat16)
         for r in range(ROWS)], axis=1)
    dnums = (((0,), (0,)), ((), ()))
    acc = lax.dot_general(ohT_i, Ti_ref[...], dnums,
                          preferred_element_type=jnp.float32)
    acc = acc + lax.dot_general(ohT_j, Tj_ref[...], dnums,
                                preferred_element_type=jnp.float32)
    acc = acc + lax.dot_general(rbf_ref[...].astype(jnp.bfloat16),
                                Wf_ref[...], dnums,
                                preferred_element_type=jnp.float32)
    acc = acc + b_ref[...]
    out_ref[...] = acc * jax.nn.sigmoid(acc)


def _tc_fused(type_i, type_j, rbf, Ti, Tj, Wf, b):
    return pl.pallas_call(
        _tc_body,
        grid=(GRID,),
        in_specs=[
            pl.BlockSpec((1, ROWS, GROUP), lambda i: (i, 0, 0)),
            pl.BlockSpec((1, ROWS, GROUP), lambda i: (i, 0, 0)),
            pl.BlockSpec((N_RBF, BLK), lambda i: (0, i)),
            pl.BlockSpec((EMBED, EMBED), lambda i: (0, 0)),
            pl.BlockSpec((EMBED, EMBED), lambda i: (0, 0)),
            pl.BlockSpec((N_RBF, EMBED), lambda i: (0, 0)),
            pl.BlockSpec((1, EMBED), lambda i: (0, 0)),
        ],
        out_specs=pl.BlockSpec((BLK, EMBED), lambda i: (i, 0)),
        out_shape=jax.ShapeDtypeStruct((N_EDGES, EMBED), jnp.float32),
        compiler_params=pltpu.CompilerParams(
            dimension_semantics=("parallel",)),
    )(type_i, type_j, rbf, Ti, Tj, Wf, b)


def kernel(rbf, species, idx_i, idx_j, embedding_vect, W_rbf, W_concat,
           b_concat):
    species = species.astype(jnp.int32)
    idx_i = idx_i.astype(jnp.int32)
    idx_j = idx_j.astype(jnp.int32)

    type_i, type_j = _make_sc_type_gather()(species, idx_i, idx_j)

    half = W_concat.shape[0] // 4
    Ti = jnp.pad(embedding_vect @ W_concat[:half],
                 ((0, EMBED - N_SPECIES), (0, 0))).astype(jnp.bfloat16)
    Tj = jnp.pad(embedding_vect @ W_concat[half:2 * half],
                 ((0, EMBED - N_SPECIES), (0, 0))).astype(jnp.bfloat16)
    Wf = (W_rbf @ W_concat[2 * half:]).astype(jnp.bfloat16)

    return _tc_fused(type_i.reshape(GRID, ROWS, GROUP),
                     type_j.reshape(GRID, ROWS, GROUP),
                     rbf.T, Ti, Tj, Wf, b_concat.reshape(1, EMBED))


# speedup vs baseline: 38.3528x; 1.0380x over previous
import functools

import jax
import jax.numpy as jnp
from jax import lax
from jax.experimental import pallas as pl
from jax.experimental.pallas import tpu as pltpu
from jax.experimental.pallas import tpu_sc as plsc

N_EDGES = 320000
N_NODES = 10000
N_RBF = 6
EMBED = 128
N_SPECIES = 100

SC_CORES = 2
SC_SUBCORES = 16
SC_LANES = 16
NW = SC_CORES * SC_SUBCORES
CHUNK = N_EDGES // NW

GROUP = 128
ROWS = 100
BLK = ROWS * GROUP
N_GROUPS = N_EDGES // GROUP
GRID = N_EDGES // BLK

@functools.cache
def _make_sc_type_gather():
    mesh = plsc.VectorSubcoreMesh(
        core_axis_name="c", subcore_axis_name="s",
        num_cores=SC_CORES, num_subcores=SC_SUBCORES)

    @functools.partial(
        pl.kernel,
        out_type=(jax.ShapeDtypeStruct((N_EDGES,), jnp.int32),
                  jax.ShapeDtypeStruct((N_EDGES,), jnp.int32)),
        mesh=mesh,
        scratch_types=[pltpu.VMEM((N_NODES,), jnp.int32),
                       pltpu.VMEM((CHUNK,), jnp.int32),
                       pltpu.VMEM((CHUNK,), jnp.int32)],
        compiler_params=pltpu.CompilerParams(needs_layout_passes=False),
    )
    def _sc_type_gather(species_hbm, idx_i_hbm, idx_j_hbm, type_i_hbm,
                        type_j_hbm, species_v, idx_v, out_v):
        wid = lax.axis_index("s") * SC_CORES + lax.axis_index("c")
        base = wid * CHUNK
        pltpu.sync_copy(species_hbm, species_v)
        for src_hbm, dst_hbm in ((idx_i_hbm, type_i_hbm),
                                 (idx_j_hbm, type_j_hbm)):
            pltpu.sync_copy(src_hbm.at[pl.ds(base, CHUNK)], idx_v)

            def body(k, _):
                iv = idx_v[pl.ds(k * SC_LANES, SC_LANES)]
                out_v[pl.ds(k * SC_LANES, SC_LANES)] = plsc.load_gather(
                    species_v, [iv])
                return 0

            lax.fori_loop(0, CHUNK // SC_LANES, body, 0)
            pltpu.sync_copy(out_v, dst_hbm.at[pl.ds(base, CHUNK)])

    return _sc_type_gather


def _tc_body(ti_ref, tj_ref, rbf_ref, Ti_ref, Tj_ref, Wf_ref, b_ref,
             out_ref):
    cls = lax.broadcasted_iota(jnp.int32, (EMBED, GROUP), 0)
    ohT_i = jnp.concatenate(
        [(ti_ref[0, r:r + 1, :] == cls).astype(jnp.bfloat16)
         for r in range(ROWS)], axis=1)
    ohT_j = jnp.concatenate(
        [(tj_ref[0, r:r + 1, :] == cls).astype(jnp.bfloat16)
         for r in range(ROWS)], axis=1)
    dnums = (((0,), (0,)), ((), ()))
    acc = lax.dot_general(ohT_i, Ti_ref[...], dnums,
                          preferred_element_type=jnp.float32)
    acc = acc + lax.dot_general(ohT_j, Tj_ref[...], dnums,
                                preferred_element_type=jnp.float32)
    acc = acc + lax.dot_general(rbf_ref[...].astype(jnp.bfloat16),
                                Wf_ref[...], dnums,
                                preferred_element_type=jnp.float32)
    acc = acc + b_ref[...]
    out_ref[...] = acc * jax.nn.sigmoid(acc)


def _tc_fused(type_i, type_j, rbf, Ti, Tj, Wf, b):
    return pl.pallas_call(
        _tc_body,
        grid=(GRID,),
        in_specs=[
            pl.BlockSpec((1, ROWS, GROUP), lambda i: (i, 0, 0)),
            pl.BlockSpec((1, ROWS, GROUP), lambda i: (i, 0, 0)),
            pl.BlockSpec((N_RBF, BLK), lambda i: (0, i)),
            pl.BlockSpec((EMBED, EMBED), lambda i: (0, 0)),
            pl.BlockSpec((EMBED, EMBED), lambda i: (0, 0)),
            pl.BlockSpec((N_RBF, EMBED), lambda i: (0, 0)),
            pl.BlockSpec((1, EMBED), lambda i: (0, 0)),
        ],
        out_specs=pl.BlockSpec((BLK, EMBED), lambda i: (i, 0)),
        out_shape=jax.ShapeDtypeStruct((N_EDGES, EMBED), jnp.float32),
        compiler_params=pltpu.CompilerParams(
            dimension_semantics=("parallel",)),
    )(type_i, type_j, rbf, Ti, Tj, Wf, b)


def kernel(rbf, species, idx_i, idx_j, embedding_vect, W_rbf, W_concat,
           b_concat):
    species = species.astype(jnp.int32)
    idx_i = idx_i.astype(jnp.int32)
    idx_j = idx_j.astype(jnp.int32)

    type_i, type_j = _make_sc_type_gather()(species, idx_i, idx_j)

    half = W_concat.shape[0] // 4
    Ti = jnp.pad(embedding_vect @ W_concat[:half],
                 ((0, EMBED - N_SPECIES), (0, 0))).astype(jnp.bfloat16)
    Tj = jnp.pad(embedding_vect @ W_concat[half:2 * half],
                 ((0, EMBED - N_SPECIES), (0, 0))).astype(jnp.bfloat16)
    Wf = (W_rbf @ W_concat[2 * half:]).astype(jnp.bfloat16)

    return _tc_fused(type_i.reshape(GRID, ROWS, GROUP),
                     type_j.reshape(GRID, ROWS, GROUP),
                     rbf.T, Ti, Tj, Wf, b_concat.reshape(1, EMBED))


# speedup vs baseline: 38.5814x; 1.0060x over previous
import functools

import jax
import jax.numpy as jnp
from jax import lax
from jax.experimental import pallas as pl
from jax.experimental.pallas import tpu as pltpu
from jax.experimental.pallas import tpu_sc as plsc

N_EDGES = 320000
N_NODES = 10000
N_RBF = 6
EMBED = 128
N_SPECIES = 100

SC_CORES = 2
SC_SUBCORES = 16
SC_LANES = 16
NW = SC_CORES * SC_SUBCORES
CHUNK = N_EDGES // NW

GROUP = 128
ROWS = 125
BLK = ROWS * GROUP
N_GROUPS = N_EDGES // GROUP
GRID = N_EDGES // BLK

@functools.cache
def _make_sc_type_gather():
    mesh = plsc.VectorSubcoreMesh(
        core_axis_name="c", subcore_axis_name="s",
        num_cores=SC_CORES, num_subcores=SC_SUBCORES)

    @functools.partial(
        pl.kernel,
        out_type=(jax.ShapeDtypeStruct((N_EDGES,), jnp.int32),
                  jax.ShapeDtypeStruct((N_EDGES,), jnp.int32)),
        mesh=mesh,
        scratch_types=[pltpu.VMEM((N_NODES,), jnp.int32),
                       pltpu.VMEM((CHUNK,), jnp.int32),
                       pltpu.VMEM((CHUNK,), jnp.int32)],
        compiler_params=pltpu.CompilerParams(needs_layout_passes=False),
    )
    def _sc_type_gather(species_hbm, idx_i_hbm, idx_j_hbm, type_i_hbm,
                        type_j_hbm, species_v, idx_v, out_v):
        wid = lax.axis_index("s") * SC_CORES + lax.axis_index("c")
        base = wid * CHUNK
        pltpu.sync_copy(species_hbm, species_v)
        for src_hbm, dst_hbm in ((idx_i_hbm, type_i_hbm),
                                 (idx_j_hbm, type_j_hbm)):
            pltpu.sync_copy(src_hbm.at[pl.ds(base, CHUNK)], idx_v)

            def body(k, _):
                iv = idx_v[pl.ds(k * SC_LANES, SC_LANES)]
                out_v[pl.ds(k * SC_LANES, SC_LANES)] = plsc.load_gather(
                    species_v, [iv])
                return 0

            lax.fori_loop(0, CHUNK // SC_LANES, body, 0)
            pltpu.sync_copy(out_v, dst_hbm.at[pl.ds(base, CHUNK)])

    return _sc_type_gather


def _tc_body(ti_ref, tj_ref, rbf_ref, Ti_ref, Tj_ref, Wf_ref, b_ref,
             out_ref):
    cls = lax.broadcasted_iota(jnp.int32, (EMBED, GROUP), 0)
    ohT_i = jnp.concatenate(
        [(ti_ref[0, r:r + 1, :] == cls).astype(jnp.bfloat16)
         for r in range(ROWS)], axis=1)
    ohT_j = jnp.concatenate(
        [(tj_ref[0, r:r + 1, :] == cls).astype(jnp.bfloat16)
         for r in range(ROWS)], axis=1)
    dnums = (((0,), (0,)), ((), ()))
    acc = lax.dot_general(ohT_i, Ti_ref[...], dnums,
                          preferred_element_type=jnp.float32)
    acc = acc + lax.dot_general(ohT_j, Tj_ref[...], dnums,
                                preferred_element_type=jnp.float32)
    acc = acc + lax.dot_general(rbf_ref[...].astype(jnp.bfloat16),
                                Wf_ref[...], dnums,
                                preferred_element_type=jnp.float32)
    acc = acc + b_ref[...]
    out_ref[...] = acc * jax.nn.sigmoid(acc)


def _tc_fused(type_i, type_j, rbf, Ti, Tj, Wf, b):
    return pl.pallas_call(
        _tc_body,
        grid=(GRID,),
        in_specs=[
            pl.BlockSpec((1, ROWS, GROUP), lambda i: (i, 0, 0)),
            pl.BlockSpec((1, ROWS, GROUP), lambda i: (i, 0, 0)),
            pl.BlockSpec((N_RBF, BLK), lambda i: (0, i)),
            pl.BlockSpec((EMBED, EMBED), lambda i: (0, 0)),
            pl.BlockSpec((EMBED, EMBED), lambda i: (0, 0)),
            pl.BlockSpec((N_RBF, EMBED), lambda i: (0, 0)),
            pl.BlockSpec((1, EMBED), lambda i: (0, 0)),
        ],
        out_specs=pl.BlockSpec((BLK, EMBED), lambda i: (i, 0)),
        out_shape=jax.ShapeDtypeStruct((N_EDGES, EMBED), jnp.float32),
        compiler_params=pltpu.CompilerParams(
            dimension_semantics=("parallel",)),
    )(type_i, type_j, rbf, Ti, Tj, Wf, b)


def kernel(rbf, species, idx_i, idx_j, embedding_vect, W_rbf, W_concat,
           b_concat):
    species = species.astype(jnp.int32)
    idx_i = idx_i.astype(jnp.int32)
    idx_j = idx_j.astype(jnp.int32)

    type_i, type_j = _make_sc_type_gather()(species, idx_i, idx_j)

    half = W_concat.shape[0] // 4
    Ti = jnp.pad(embedding_vect @ W_concat[:half],
                 ((0, EMBED - N_SPECIES), (0, 0))).astype(jnp.bfloat16)
    Tj = jnp.pad(embedding_vect @ W_concat[half:2 * half],
                 ((0, EMBED - N_SPECIES), (0, 0))).astype(jnp.bfloat16)
    Wf = (W_rbf @ W_concat[2 * half:]).astype(jnp.bfloat16)

    return _tc_fused(type_i.reshape(GRID, ROWS, GROUP),
                     type_j.reshape(GRID, ROWS, GROUP),
                     rbf.T, Ti, Tj, Wf, b_concat.reshape(1, EMBED))


# speedup vs baseline: 40.4578x; 1.0486x over previous
import functools

import jax
import jax.numpy as jnp
from jax import lax
from jax.experimental import pallas as pl
from jax.experimental.pallas import tpu as pltpu
from jax.experimental.pallas import tpu_sc as plsc

N_EDGES = 320000
N_NODES = 10000
N_RBF = 6
EMBED = 128
N_SPECIES = 100

SC_CORES = 2
SC_SUBCORES = 16
SC_LANES = 16
NW = SC_CORES * SC_SUBCORES
CHUNK = N_EDGES // NW

GROUP = 128
ROWS = 125
BLK = ROWS * GROUP
N_GROUPS = N_EDGES // GROUP
GRID = N_EDGES // BLK

@functools.cache
def _make_sc_type_gather():
    mesh = plsc.VectorSubcoreMesh(
        core_axis_name="c", subcore_axis_name="s",
        num_cores=SC_CORES, num_subcores=SC_SUBCORES)

    @functools.partial(
        pl.kernel,
        out_type=(jax.ShapeDtypeStruct((N_EDGES,), jnp.int32),
                  jax.ShapeDtypeStruct((N_EDGES,), jnp.int32)),
        mesh=mesh,
        scratch_types=[pltpu.VMEM((N_NODES,), jnp.int32),
                       pltpu.VMEM((CHUNK,), jnp.int32),
                       pltpu.VMEM((CHUNK,), jnp.int32)],
        compiler_params=pltpu.CompilerParams(needs_layout_passes=False),
    )
    def _sc_type_gather(species_hbm, idx_i_hbm, idx_j_hbm, type_i_hbm,
                        type_j_hbm, species_v, idx_v, out_v):
        wid = lax.axis_index("s") * SC_CORES + lax.axis_index("c")
        base = wid * CHUNK
        pltpu.sync_copy(species_hbm, species_v)
        for src_hbm, dst_hbm in ((idx_i_hbm, type_i_hbm),
                                 (idx_j_hbm, type_j_hbm)):
            pltpu.sync_copy(src_hbm.at[pl.ds(base, CHUNK)], idx_v)

            @plsc.parallel_loop(0, CHUNK, step=SC_LANES, unroll=8)
            def body(k):
                iv = idx_v[pl.ds(k, SC_LANES)]
                out_v[pl.ds(k, SC_LANES)] = plsc.load_gather(
                    species_v, [iv])

            pltpu.sync_copy(out_v, dst_hbm.at[pl.ds(base, CHUNK)])

    return _sc_type_gather


def _tc_body(ti_ref, tj_ref, rbf_ref, Ti_ref, Tj_ref, Wf_ref, b_ref,
             out_ref):
    cls = lax.broadcasted_iota(jnp.int32, (EMBED, GROUP), 0)
    ohT_i = jnp.concatenate(
        [(ti_ref[0, r:r + 1, :] == cls).astype(jnp.bfloat16)
         for r in range(ROWS)], axis=1)
    ohT_j = jnp.concatenate(
        [(tj_ref[0, r:r + 1, :] == cls).astype(jnp.bfloat16)
         for r in range(ROWS)], axis=1)
    dnums = (((0,), (0,)), ((), ()))
    acc = lax.dot_general(ohT_i, Ti_ref[...], dnums,
                          preferred_element_type=jnp.float32)
    acc = acc + lax.dot_general(ohT_j, Tj_ref[...], dnums,
                                preferred_element_type=jnp.float32)
    acc = acc + lax.dot_general(rbf_ref[...].astype(jnp.bfloat16),
                                Wf_ref[...], dnums,
                                preferred_element_type=jnp.float32)
    acc = acc + b_ref[...]
    out_ref[...] = acc * jax.nn.sigmoid(acc)


def _tc_fused(type_i, type_j, rbf, Ti, Tj, Wf, b):
    return pl.pallas_call(
        _tc_body,
        grid=(GRID,),
        in_specs=[
            pl.BlockSpec((1, ROWS, GROUP), lambda i: (i, 0, 0)),
            pl.BlockSpec((1, ROWS, GROUP), lambda i: (i, 0, 0)),
            pl.BlockSpec((N_RBF, BLK), lambda i: (0, i)),
            pl.BlockSpec((EMBED, EMBED), lambda i: (0, 0)),
            pl.BlockSpec((EMBED, EMBED), lambda i: (0, 0)),
            pl.BlockSpec((N_RBF, EMBED), lambda i: (0, 0)),
            pl.BlockSpec((1, EMBED), lambda i: (0, 0)),
        ],
        out_specs=pl.BlockSpec((BLK, EMBED), lambda i: (i, 0)),
        out_shape=jax.ShapeDtypeStruct((N_EDGES, EMBED), jnp.float32),
        compiler_params=pltpu.CompilerParams(
            dimension_semantics=("parallel",)),
    )(type_i, type_j, rbf, Ti, Tj, Wf, b)


def kernel(rbf, species, idx_i, idx_j, embedding_vect, W_rbf, W_concat,
           b_concat):
    species = species.astype(jnp.int32)
    idx_i = idx_i.astype(jnp.int32)
    idx_j = idx_j.astype(jnp.int32)

    type_i, type_j = _make_sc_type_gather()(species, idx_i, idx_j)

    half = W_concat.shape[0] // 4
    Ti = jnp.pad(embedding_vect @ W_concat[:half],
                 ((0, EMBED - N_SPECIES), (0, 0))).astype(jnp.bfloat16)
    Tj = jnp.pad(embedding_vect @ W_concat[half:2 * half],
                 ((0, EMBED - N_SPECIES), (0, 0))).astype(jnp.bfloat16)
    Wf = (W_rbf @ W_concat[2 * half:]).astype(jnp.bfloat16)

    return _tc_fused(type_i.reshape(GRID, ROWS, GROUP),
                     type_j.reshape(GRID, ROWS, GROUP),
                     rbf.T, Ti, Tj, Wf, b_concat.reshape(1, EMBED))


# speedup vs baseline: 40.5248x; 1.0017x over previous
import functools

import jax
import jax.numpy as jnp
from jax import lax
from jax.experimental import pallas as pl
from jax.experimental.pallas import tpu as pltpu
from jax.experimental.pallas import tpu_sc as plsc

N_EDGES = 320000
N_NODES = 10000
N_RBF = 6
EMBED = 128
N_SPECIES = 100

SC_CORES = 2
SC_SUBCORES = 16
SC_LANES = 16
NW = SC_CORES * SC_SUBCORES
CHUNK = N_EDGES // NW

GROUP = 128
ROWS = 250
BLK = ROWS * GROUP
N_GROUPS = N_EDGES // GROUP
GRID = N_EDGES // BLK

@functools.cache
def _make_sc_type_gather():
    mesh = plsc.VectorSubcoreMesh(
        core_axis_name="c", subcore_axis_name="s",
        num_cores=SC_CORES, num_subcores=SC_SUBCORES)

    @functools.partial(
        pl.kernel,
        out_type=(jax.ShapeDtypeStruct((N_EDGES,), jnp.int32),
                  jax.ShapeDtypeStruct((N_EDGES,), jnp.int32)),
        mesh=mesh,
        scratch_types=[pltpu.VMEM((N_NODES,), jnp.int32),
                       pltpu.VMEM((CHUNK,), jnp.int32),
                       pltpu.VMEM((CHUNK,), jnp.int32)],
        compiler_params=pltpu.CompilerParams(needs_layout_passes=False),
    )
    def _sc_type_gather(species_hbm, idx_i_hbm, idx_j_hbm, type_i_hbm,
                        type_j_hbm, species_v, idx_v, out_v):
        wid = lax.axis_index("s") * SC_CORES + lax.axis_index("c")
        base = wid * CHUNK
        pltpu.sync_copy(species_hbm, species_v)
        for src_hbm, dst_hbm in ((idx_i_hbm, type_i_hbm),
                                 (idx_j_hbm, type_j_hbm)):
            pltpu.sync_copy(src_hbm.at[pl.ds(base, CHUNK)], idx_v)

            @plsc.parallel_loop(0, CHUNK, step=SC_LANES, unroll=8)
            def body(k):
                iv = idx_v[pl.ds(k, SC_LANES)]
                out_v[pl.ds(k, SC_LANES)] = plsc.load_gather(
                    species_v, [iv])

            pltpu.sync_copy(out_v, dst_hbm.at[pl.ds(base, CHUNK)])

    return _sc_type_gather


def _tc_body(ti_ref, tj_ref, rbf_ref, Ti_ref, Tj_ref, Wf_ref, b_ref,
             out_ref):
    cls = lax.broadcasted_iota(jnp.int32, (EMBED, GROUP), 0)
    ohT_i = jnp.concatenate(
        [(ti_ref[0, r:r + 1, :] == cls).astype(jnp.bfloat16)
         for r in range(ROWS)], axis=1)
    ohT_j = jnp.concatenate(
        [(tj_ref[0, r:r + 1, :] == cls).astype(jnp.bfloat16)
         for r in range(ROWS)], axis=1)
    dnums = (((0,), (0,)), ((), ()))
    acc = lax.dot_general(ohT_i, Ti_ref[...], dnums,
                          preferred_element_type=jnp.float32)
    acc = acc + lax.dot_general(ohT_j, Tj_ref[...], dnums,
                                preferred_element_type=jnp.float32)
    acc = acc + lax.dot_general(rbf_ref[...].astype(jnp.bfloat16),
                                Wf_ref[...], dnums,
                                preferred_element_type=jnp.float32)
    acc = acc + b_ref[...]
    out_ref[...] = acc * jax.nn.sigmoid(acc)


def _tc_fused(type_i, type_j, rbf, Ti, Tj, Wf, b):
    return pl.pallas_call(
        _tc_body,
        grid=(GRID,),
        in_specs=[
            pl.BlockSpec((1, ROWS, GROUP), lambda i: (i, 0, 0)),
            pl.BlockSpec((1, ROWS, GROUP), lambda i: (i, 0, 0)),
            pl.BlockSpec((N_RBF, BLK), lambda i: (0, i)),
            pl.BlockSpec((EMBED, EMBED), lambda i: (0, 0)),
            pl.BlockSpec((EMBED, EMBED), lambda i: (0, 0)),
            pl.BlockSpec((N_RBF, EMBED), lambda i: (0, 0)),
            pl.BlockSpec((1, EMBED), lambda i: (0, 0)),
        ],
        out_specs=pl.BlockSpec((BLK, EMBED), lambda i: (i, 0)),
        out_shape=jax.ShapeDtypeStruct((N_EDGES, EMBED), jnp.float32),
        compiler_params=pltpu.CompilerParams(
            dimension_semantics=("parallel",)),
    )(type_i, type_j, rbf, Ti, Tj, Wf, b)


def kernel(rbf, species, idx_i, idx_j, embedding_vect, W_rbf, W_concat,
           b_concat):
    species = species.astype(jnp.int32)
    idx_i = idx_i.astype(jnp.int32)
    idx_j = idx_j.astype(jnp.int32)

    type_i, type_j = _make_sc_type_gather()(species, idx_i, idx_j)

    half = W_concat.shape[0] // 4
    Ti = jnp.pad(embedding_vect @ W_concat[:half],
                 ((0, EMBED - N_SPECIES), (0, 0))).astype(jnp.bfloat16)
    Tj = jnp.pad(embedding_vect @ W_concat[half:2 * half],
                 ((0, EMBED - N_SPECIES), (0, 0))).astype(jnp.bfloat16)
    Wf = (W_rbf @ W_concat[2 * half:]).astype(jnp.bfloat16)

    return _tc_fused(type_i.reshape(GRID, ROWS, GROUP),
                     type_j.reshape(GRID, ROWS, GROUP),
                     rbf.T, Ti, Tj, Wf, b_concat.reshape(1, EMBED))


# speedup vs baseline: 41.0630x; 1.0133x over previous
import functools

import jax
import jax.numpy as jnp
from jax import lax
from jax.experimental import pallas as pl
from jax.experimental.pallas import tpu as pltpu
from jax.experimental.pallas import tpu_sc as plsc

N_EDGES = 320000
N_NODES = 10000
N_RBF = 6
EMBED = 128
N_SPECIES = 100

SC_CORES = 2
SC_SUBCORES = 16
SC_LANES = 16
NW = SC_CORES * SC_SUBCORES
CHUNK = N_EDGES // NW

GROUP = 128
ROWS = 125
BLK = ROWS * GROUP
N_GROUPS = N_EDGES // GROUP
GRID = N_EDGES // BLK

@functools.cache
def _make_sc_type_gather():
    mesh = plsc.VectorSubcoreMesh(
        core_axis_name="c", subcore_axis_name="s",
        num_cores=SC_CORES, num_subcores=SC_SUBCORES)

    @functools.partial(
        pl.kernel,
        out_type=jax.ShapeDtypeStruct((N_EDGES,), jnp.int32),
        mesh=mesh,
        scratch_types=[pltpu.VMEM((N_NODES,), jnp.int32),
                       pltpu.VMEM((CHUNK,), jnp.int32),
                       pltpu.VMEM((CHUNK,), jnp.int32),
                       pltpu.VMEM((CHUNK,), jnp.int32)],
        compiler_params=pltpu.CompilerParams(needs_layout_passes=False),
    )
    def _sc_type_gather(species_hbm, idx_i_hbm, idx_j_hbm, types_hbm,
                        species_v, idx_i_v, idx_j_v, out_v):
        wid = lax.axis_index("s") * SC_CORES + lax.axis_index("c")
        base = wid * CHUNK
        pltpu.sync_copy(species_hbm, species_v)
        pltpu.sync_copy(idx_i_hbm.at[pl.ds(base, CHUNK)], idx_i_v)
        pltpu.sync_copy(idx_j_hbm.at[pl.ds(base, CHUNK)], idx_j_v)

        @plsc.parallel_loop(0, CHUNK, step=SC_LANES, unroll=8)
        def body(k):
            ti = plsc.load_gather(species_v, [idx_i_v[pl.ds(k, SC_LANES)]])
            tj = plsc.load_gather(species_v, [idx_j_v[pl.ds(k, SC_LANES)]])
            out_v[pl.ds(k, SC_LANES)] = ti | (tj << 8)

        pltpu.sync_copy(out_v, types_hbm.at[pl.ds(base, CHUNK)])

    return _sc_type_gather


def _tc_body(t_ref, rbf_ref, Ti_ref, Tj_ref, Wf_ref, b_ref, out_ref):
    cls = lax.broadcasted_iota(jnp.int32, (EMBED, GROUP), 0)
    ohT_i = jnp.concatenate(
        [((t_ref[0, r:r + 1, :] & 0xFF) == cls).astype(jnp.bfloat16)
         for r in range(ROWS)], axis=1)
    ohT_j = jnp.concatenate(
        [((t_ref[0, r:r + 1, :] >> 8) == cls).astype(jnp.bfloat16)
         for r in range(ROWS)], axis=1)
    dnums = (((0,), (0,)), ((), ()))
    acc = lax.dot_general(ohT_i, Ti_ref[...], dnums,
                          preferred_element_type=jnp.float32)
    acc = acc + lax.dot_general(ohT_j, Tj_ref[...], dnums,
                                preferred_element_type=jnp.float32)
    acc = acc + lax.dot_general(rbf_ref[...].astype(jnp.bfloat16),
                                Wf_ref[...], dnums,
                                preferred_element_type=jnp.float32)
    acc = acc + b_ref[...]
    out_ref[...] = acc * jax.nn.sigmoid(acc)


def _tc_fused(types, rbf, Ti, Tj, Wf, b):
    return pl.pallas_call(
        _tc_body,
        grid=(GRID,),
        in_specs=[
            pl.BlockSpec((1, ROWS, GROUP), lambda i: (i, 0, 0)),
            pl.BlockSpec((N_RBF, BLK), lambda i: (0, i)),
            pl.BlockSpec((EMBED, EMBED), lambda i: (0, 0)),
            pl.BlockSpec((EMBED, EMBED), lambda i: (0, 0)),
            pl.BlockSpec((N_RBF, EMBED), lambda i: (0, 0)),
            pl.BlockSpec((1, EMBED), lambda i: (0, 0)),
        ],
        out_specs=pl.BlockSpec((BLK, EMBED), lambda i: (i, 0)),
        out_shape=jax.ShapeDtypeStruct((N_EDGES, EMBED), jnp.float32),
        compiler_params=pltpu.CompilerParams(
            dimension_semantics=("parallel",)),
    )(types, rbf, Ti, Tj, Wf, b)


def kernel(rbf, species, idx_i, idx_j, embedding_vect, W_rbf, W_concat,
           b_concat):
    species = species.astype(jnp.int32)
    idx_i = idx_i.astype(jnp.int32)
    idx_j = idx_j.astype(jnp.int32)

    types = _make_sc_type_gather()(species, idx_i, idx_j)

    half = W_concat.shape[0] // 4
    Ti = jnp.pad(embedding_vect @ W_concat[:half],
                 ((0, EMBED - N_SPECIES), (0, 0))).astype(jnp.bfloat16)
    Tj = jnp.pad(embedding_vect @ W_concat[half:2 * half],
                 ((0, EMBED - N_SPECIES), (0, 0))).astype(jnp.bfloat16)
    Wf = (W_rbf @ W_concat[2 * half:]).astype(jnp.bfloat16)

    return _tc_fused(types.reshape(GRID, ROWS, GROUP),
                     rbf.T, Ti, Tj, Wf, b_concat.reshape(1, EMBED))
